# baseline (speedup 1.0000x reference)
import jax
import jax.numpy as jnp
from jax.experimental import pallas as pl
from jax.experimental.pallas import tpu as pltpu

N_CATEGORY = 20
N_TYPE_PER_CAT = 10
N_VARIANT_PER_TYPE = 4
N_SPATIAL = 20
D = 320
TOTAL = 1040
VCH = 4
VMAJ = N_CATEGORY // VCH
VROWS = 800 // VCH


def _concat_body(cat_ref, typ_ref, var_ref, spa_ref, out_ref,
                 bcat, btyp, bvar, bspa, ot,
                 s_cat, s_typ, s_spa, s_out, *s_var):
    c_var = [
        pltpu.make_async_copy(
            var_ref.at[pl.ds(k * VMAJ, VMAJ)],
            bvar.at[pl.ds(k * VMAJ, VMAJ)],
            s_var[k],
        )
        for k in range(VCH)
    ]
    c_typ = pltpu.make_async_copy(typ_ref, btyp, s_typ)
    c_cat = pltpu.make_async_copy(cat_ref, bcat, s_cat)
    c_spa = pltpu.make_async_copy(spa_ref, bspa, s_spa)
    for c in c_var:
        c.start()
    c_typ.start()
    c_cat.start()
    c_spa.start()

    c_cat.wait()
    ot[:, 0:20] = jnp.transpose(bcat[...])
    c_typ.wait()
    for i in range(N_CATEGORY):
        ot[:, 20 + 10 * i:30 + 10 * i] = jnp.transpose(btyp[:, i, :])
    c_spa.wait()
    ot[:, 1020:1040] = jnp.transpose(bspa[...])
    for k in range(VCH):
        c_var[k].wait()
        ot[:, 220 + k * VROWS:220 + (k + 1) * VROWS] = jnp.transpose(
            bvar[k * VMAJ:(k + 1) * VMAJ].reshape(VROWS, D))

    outs = []
    for a in range(3):
        w = 128 if a < 2 else 64
        o = pltpu.make_async_copy(
            ot.at[pl.ds(128 * a, w)], out_ref.at[pl.ds(128 * a, w)], s_out)
        o.start()
        outs.append(o)
    for o in outs:
        o.wait()


def kernel(category_codes, type_codes, variant_codes, spatial_codes):
    out_t = pl.pallas_call(
        _concat_body,
        out_shape=jax.ShapeDtypeStruct((D, TOTAL), jnp.float32),
        in_specs=[pl.BlockSpec(memory_space=pl.ANY)] * 4,
        out_specs=pl.BlockSpec(memory_space=pl.ANY),
        scratch_shapes=[
            pltpu.VMEM((N_CATEGORY, D), jnp.float32),
            pltpu.VMEM((N_TYPE_PER_CAT, N_CATEGORY, D), jnp.float32),
            pltpu.VMEM((N_CATEGORY, N_TYPE_PER_CAT, N_VARIANT_PER_TYPE, D),
                       jnp.float32),
            pltpu.VMEM((N_SPATIAL, D), jnp.float32),
            pltpu.VMEM((D, TOTAL), jnp.float32),
        ] + [pltpu.SemaphoreType.DMA] * (4 + VCH),
    )(category_codes, type_codes.transpose(1, 0, 2), variant_codes,
      spatial_codes)
    return out_t.T


# speedup vs baseline: 1.8988x; 1.8988x over previous
import jax
import jax.numpy as jnp
from jax.experimental import pallas as pl
from jax.experimental.pallas import tpu as pltpu

N_CATEGORY = 20
N_TYPE_PER_CAT = 10
N_VARIANT_PER_TYPE = 4
N_SPATIAL = 20
D = 320
TOTAL = 1040


def _concat_body(cat_ref, typ_ref, var_ref, spa_ref, out_ref,
                 bcat, btyp, bvar, bspa, obuf, ot,
                 s_cat, s_typ, s_var, s_spa, s_out):
    c_cat = pltpu.make_async_copy(cat_ref, bcat, s_cat)
    c_typ = pltpu.make_async_copy(typ_ref, btyp, s_typ)
    c_var = pltpu.make_async_copy(var_ref, bvar, s_var)
    c_spa = pltpu.make_async_copy(spa_ref, bspa, s_spa)
    for c in (c_var, c_typ, c_cat, c_spa):
        c.start()

    c_cat.wait()
    obuf[0:20] = bcat[...]
    c_typ.wait()
    for i in range(N_CATEGORY):
        obuf[20 + 10 * i:30 + 10 * i] = btyp[:, i, :]
    c_var.wait()
    obuf[220:1020] = bvar[...].reshape(800, D)
    c_spa.wait()
    obuf[1020:1040] = bspa[...]

    outs = []
    for a in range(3):
        w = 128 if a < 2 else 64
        ot[128 * a:128 * a + w, :] = jnp.transpose(
            obuf[:, 128 * a:128 * a + w])
        o = pltpu.make_async_copy(
            ot.at[pl.ds(128 * a, w)], out_ref.at[pl.ds(128 * a, w)], s_out)
        o.start()
        outs.append(o)
    for o in outs:
        o.wait()


def kernel(category_codes, type_codes, variant_codes, spatial_codes):
    out_t = pl.pallas_call(
        _concat_body,
        out_shape=jax.ShapeDtypeStruct((D, TOTAL), jnp.float32),
        in_specs=[pl.BlockSpec(memory_space=pl.ANY)] * 4,
        out_specs=pl.BlockSpec(memory_space=pl.ANY),
        scratch_shapes=[
            pltpu.VMEM((N_CATEGORY, D), jnp.float32),
            pltpu.VMEM((N_TYPE_PER_CAT, N_CATEGORY, D), jnp.float32),
            pltpu.VMEM((N_CATEGORY, N_TYPE_PER_CAT, N_VARIANT_PER_TYPE, D),
                       jnp.float32),
            pltpu.VMEM((N_SPATIAL, D), jnp.float32),
            pltpu.VMEM((TOTAL, D), jnp.float32),
            pltpu.VMEM((D, TOTAL), jnp.float32),
        ] + [pltpu.SemaphoreType.DMA] * 5,
    )(category_codes, type_codes.transpose(1, 0, 2), variant_codes,
      spatial_codes)
    return out_t.T
